# Initial kernel scaffold; baseline (speedup 1.0000x reference)
import jax
import jax.numpy as jnp
from jax.experimental import pallas as pl


def kernel(x, scale_factors, W_act, b_act, W_comp, b_comp, W_reg, b_reg):
    raise NotImplementedError("write your pallas kernel here")


# speedup vs baseline: 1.1725x; 1.1725x over previous
import functools

import jax
import jax.numpy as jnp
from jax.experimental import pallas as pl

_NUM_SAMPLES = 1024
_NUM_SEG = 9
_FEAT = 3072
_NUM_CLASSES = 20
_NOUT = 128


def _fused_kernel(x_ref, sf_ref, w_ref, b_ref, out_ref):
    sf = sf_ref[...]
    start = (x_ref[:, 0, :] + x_ref[:, 1, :]) * (sf[:, 0:1] * 0.5)
    course = (x_ref[:, 2, :] + x_ref[:, 3, :] + x_ref[:, 4, :]
              + x_ref[:, 5, :] + x_ref[:, 6, :]) * 0.2
    end = (x_ref[:, 7, :] + x_ref[:, 8, :]) * (sf[:, 1:2] * 0.5)
    acc = jnp.dot(start, w_ref[0], preferred_element_type=jnp.float32)
    acc += jnp.dot(course, w_ref[1], preferred_element_type=jnp.float32)
    acc += jnp.dot(end, w_ref[2], preferred_element_type=jnp.float32)
    out_ref[...] = acc + b_ref[...]


@functools.partial(jax.jit, static_argnames=("block",))
def _run(x3, sf, wcat, bcat, block=64):
    grid = _NUM_SAMPLES // block
    return pl.pallas_call(
        _fused_kernel,
        grid=(grid,),
        in_specs=[
            pl.BlockSpec((block, _NUM_SEG, _FEAT), lambda i: (i, 0, 0)),
            pl.BlockSpec((block, 2), lambda i: (i, 0)),
            pl.BlockSpec((3, _FEAT, _NOUT), lambda i: (0, 0, 0)),
            pl.BlockSpec((1, _NOUT), lambda i: (0, 0)),
        ],
        out_specs=pl.BlockSpec((block, _NOUT), lambda i: (i, 0)),
        out_shape=jax.ShapeDtypeStruct((_NUM_SAMPLES, _NOUT), jnp.float32),
    )(x3, sf, wcat, bcat)


def kernel(x, scale_factors, W_act, b_act, W_comp, b_comp, W_reg, b_reg):
    F = _FEAT
    x3 = x.reshape(_NUM_SAMPLES, _NUM_SEG, F)
    wcat = jnp.zeros((3, F, _NOUT), jnp.float32)
    wcat = wcat.at[1, :, 0:21].set(W_act.T)
    for p in range(3):
        wcat = wcat.at[p, :, 21:41].set(W_comp[:, p * F:(p + 1) * F].T)
        wcat = wcat.at[p, :, 41:81].set(W_reg[:, p * F:(p + 1) * F].T)
    bcat = jnp.zeros((1, _NOUT), jnp.float32)
    bcat = bcat.at[0, 0:21].set(b_act)
    bcat = bcat.at[0, 21:41].set(b_comp)
    bcat = bcat.at[0, 41:81].set(b_reg)
    out = _run(x3, scale_factors, wcat, bcat)
    activity_scores = out[:, 0:21]
    complete_scores = out[:, 21:41]
    bbox_preds = out[:, 41:81].reshape(-1, _NUM_CLASSES, 2)
    return (activity_scores, complete_scores, bbox_preds)


# speedup vs baseline: 1.8154x; 1.5484x over previous
import functools

import jax
import jax.numpy as jnp
from jax.experimental import pallas as pl

_NUM_SAMPLES = 1024
_NUM_SEG = 9
_FEAT = 3072
_NUM_CLASSES = 20

_DN = (((1,), (1,)), ((), ()))


def _dot_t(a, w):
    return jax.lax.dot_general(a, w, _DN, preferred_element_type=jnp.float32)


def _fused_kernel(x_ref, sf_ref, wa_ref, ba_ref, wc_ref, bc_ref, wr_ref, br_ref,
                  act_ref, comp_ref, reg_ref):
    F = _FEAT
    sf = sf_ref[...]
    start = (x_ref[:, 0, :] + x_ref[:, 1, :]) * (sf[:, 0:1] * 0.5)
    course = (x_ref[:, 2, :] + x_ref[:, 3, :] + x_ref[:, 4, :]
              + x_ref[:, 5, :] + x_ref[:, 6, :]) * 0.2
    end = (x_ref[:, 7, :] + x_ref[:, 8, :]) * (sf[:, 1:2] * 0.5)
    act_ref[...] = _dot_t(course, wa_ref[...]) + ba_ref[...]
    comp_ref[...] = (_dot_t(start, wc_ref[:, 0:F])
                     + _dot_t(course, wc_ref[:, F:2 * F])
                     + _dot_t(end, wc_ref[:, 2 * F:3 * F]) + bc_ref[...])
    reg_ref[...] = (_dot_t(start, wr_ref[:, 0:F])
                    + _dot_t(course, wr_ref[:, F:2 * F])
                    + _dot_t(end, wr_ref[:, 2 * F:3 * F]) + br_ref[...])


@functools.partial(jax.jit, static_argnames=("block",))
def _run(x3, sf, W_act, b_act, W_comp, b_comp, W_reg, b_reg, block=64):
    grid = _NUM_SAMPLES // block
    nw = lambda i: (0, 0)
    outs = pl.pallas_call(
        _fused_kernel,
        grid=(grid,),
        in_specs=[
            pl.BlockSpec((block, _NUM_SEG, _FEAT), lambda i: (i, 0, 0)),
            pl.BlockSpec((block, 2), lambda i: (i, 0)),
            pl.BlockSpec(W_act.shape, nw),
            pl.BlockSpec(b_act.shape, nw),
            pl.BlockSpec(W_comp.shape, nw),
            pl.BlockSpec(b_comp.shape, nw),
            pl.BlockSpec(W_reg.shape, nw),
            pl.BlockSpec(b_reg.shape, nw),
        ],
        out_specs=[
            pl.BlockSpec((block, _NUM_CLASSES + 1), lambda i: (i, 0)),
            pl.BlockSpec((block, _NUM_CLASSES), lambda i: (i, 0)),
            pl.BlockSpec((block, _NUM_CLASSES * 2), lambda i: (i, 0)),
        ],
        out_shape=[
            jax.ShapeDtypeStruct((_NUM_SAMPLES, _NUM_CLASSES + 1), jnp.float32),
            jax.ShapeDtypeStruct((_NUM_SAMPLES, _NUM_CLASSES), jnp.float32),
            jax.ShapeDtypeStruct((_NUM_SAMPLES, _NUM_CLASSES * 2), jnp.float32),
        ],
    )(x3, sf, W_act, b_act, W_comp, b_comp, W_reg, b_reg)
    return outs


def kernel(x, scale_factors, W_act, b_act, W_comp, b_comp, W_reg, b_reg):
    x3 = x.reshape(_NUM_SAMPLES, _NUM_SEG, _FEAT)
    act, comp, reg = _run(x3, scale_factors,
                          W_act, b_act.reshape(1, -1),
                          W_comp, b_comp.reshape(1, -1),
                          W_reg, b_reg.reshape(1, -1))
    return (act, comp, reg.reshape(-1, _NUM_CLASSES, 2))


# speedup vs baseline: 1.8160x; 1.0003x over previous
import functools

import jax
import jax.numpy as jnp
from jax.experimental import pallas as pl
from jax.experimental.pallas import tpu as pltpu

_NUM_SAMPLES = 1024
_NUM_SEG = 9
_FEAT = 3072
_NUM_CLASSES = 20

_DN = (((1,), (1,)), ((), ()))


def _dot_t(a, w):
    return jax.lax.dot_general(a, w, _DN, preferred_element_type=jnp.float32)


def _fused_kernel(x_ref, sf_ref, wa_ref, ba_ref, wc_ref, bc_ref, wr_ref, br_ref,
                  act_ref, comp_ref, reg_ref):
    F = _FEAT
    sf = sf_ref[...]
    start = (x_ref[:, 0, :] + x_ref[:, 1, :]) * (sf[:, 0:1] * 0.5)
    course = (x_ref[:, 2, :] + x_ref[:, 3, :] + x_ref[:, 4, :]
              + x_ref[:, 5, :] + x_ref[:, 6, :]) * 0.2
    end = (x_ref[:, 7, :] + x_ref[:, 8, :]) * (sf[:, 1:2] * 0.5)
    act_ref[...] = _dot_t(course, wa_ref[...]) + ba_ref[...]
    comp_ref[...] = (_dot_t(start, wc_ref[:, 0:F])
                     + _dot_t(course, wc_ref[:, F:2 * F])
                     + _dot_t(end, wc_ref[:, 2 * F:3 * F]) + bc_ref[...])
    reg_ref[...] = (_dot_t(start, wr_ref[:, 0:F])
                    + _dot_t(course, wr_ref[:, F:2 * F])
                    + _dot_t(end, wr_ref[:, 2 * F:3 * F]) + br_ref[...])


@functools.partial(jax.jit, static_argnames=("block",))
def _run(x3, sf, W_act, b_act, W_comp, b_comp, W_reg, b_reg, block=64):
    grid = _NUM_SAMPLES // block
    nw = lambda i: (0, 0)
    outs = pl.pallas_call(
        _fused_kernel,
        grid=(grid,),
        in_specs=[
            pl.BlockSpec((block, _NUM_SEG, _FEAT), lambda i: (i, 0, 0)),
            pl.BlockSpec((block, 2), lambda i: (i, 0)),
            pl.BlockSpec(W_act.shape, nw),
            pl.BlockSpec(b_act.shape, nw),
            pl.BlockSpec(W_comp.shape, nw),
            pl.BlockSpec(b_comp.shape, nw),
            pl.BlockSpec(W_reg.shape, nw),
            pl.BlockSpec(b_reg.shape, nw),
        ],
        out_specs=[
            pl.BlockSpec((block, _NUM_CLASSES + 1), lambda i: (i, 0)),
            pl.BlockSpec((block, _NUM_CLASSES), lambda i: (i, 0)),
            pl.BlockSpec((block, _NUM_CLASSES * 2), lambda i: (i, 0)),
        ],
        out_shape=[
            jax.ShapeDtypeStruct((_NUM_SAMPLES, _NUM_CLASSES + 1), jnp.float32),
            jax.ShapeDtypeStruct((_NUM_SAMPLES, _NUM_CLASSES), jnp.float32),
            jax.ShapeDtypeStruct((_NUM_SAMPLES, _NUM_CLASSES * 2), jnp.float32),
        ],
        compiler_params=pltpu.CompilerParams(
            dimension_semantics=("parallel",)),
    )(x3, sf, W_act, b_act, W_comp, b_comp, W_reg, b_reg)
    return outs


def kernel(x, scale_factors, W_act, b_act, W_comp, b_comp, W_reg, b_reg):
    x3 = x.reshape(_NUM_SAMPLES, _NUM_SEG, _FEAT)
    act, comp, reg = _run(x3, scale_factors,
                          W_act, b_act.reshape(1, -1),
                          W_comp, b_comp.reshape(1, -1),
                          W_reg, b_reg.reshape(1, -1))
    return (act, comp, reg.reshape(-1, _NUM_CLASSES, 2))
